# baseline (device time: 46504 ns/iter reference)
import jax
import jax.numpy as jnp
from jax import lax
from jax.experimental import pallas as pl
from jax.experimental.pallas import tpu as pltpu

N_DEV = 4


def kernel(x, w_mat, scale_x, scale_w):
    m_per, k = x.shape
    _, n = w_mat.shape
    n_per = n // N_DEV

    def body(x_ref, w_ref, sx_ref, sw_ref, out_ref,
             w4, ybuf, rbuf, send_sems, recv_sems):
        my = lax.axis_index("i")

        barrier_sem = pltpu.get_barrier_semaphore()
        for d in range(1, N_DEV):
            pl.semaphore_signal(
                barrier_sem,
                inc=1,
                device_id=((my + d) % N_DEV,),
                device_id_type=pl.DeviceIdType.MESH,
            )
        pl.semaphore_wait(barrier_sem, N_DEV - 1)

        for p in range(N_DEV):
            w4[p, :, :] = w_ref[:, p * n_per:(p + 1) * n_per]

        scale = sx_ref[0] * sw_ref[0]

        for d in range(1, N_DEV):
            p = (my + d) % N_DEV
            acc = jnp.dot(x_ref[:, :], w4[p, :, :],
                          preferred_element_type=jnp.int32)
            ybuf[d - 1, :, :] = ((acc + 1024) >> 11).astype(jnp.int16)
            rdma = pltpu.make_async_remote_copy(
                src_ref=ybuf.at[d - 1],
                dst_ref=rbuf.at[d - 1],
                send_sem=send_sems.at[d - 1],
                recv_sem=recv_sems.at[d - 1],
                device_id=(p,),
                device_id_type=pl.DeviceIdType.MESH,
            )
            rdma.start()

        acc = jnp.dot(x_ref[:, :], w4[my, :, :],
                      preferred_element_type=jnp.int32)
        out_ref[pl.ds(my * m_per, m_per), :] = acc.astype(jnp.float32) * scale

        for d in range(1, N_DEV):
            src = (my - d) % N_DEV
            waiter = pltpu.make_async_remote_copy(
                src_ref=ybuf.at[d - 1],
                dst_ref=rbuf.at[d - 1],
                send_sem=send_sems.at[d - 1],
                recv_sem=recv_sems.at[d - 1],
                device_id=((my + d) % N_DEV,),
                device_id_type=pl.DeviceIdType.MESH,
            )
            waiter.wait_recv()
            out_ref[pl.ds(src * m_per, m_per), :] = (
                rbuf[d - 1, :, :].astype(jnp.float32) * (2048.0 * scale))
            waiter.wait_send()

    return pl.pallas_call(
        body,
        out_shape=jax.ShapeDtypeStruct((N_DEV * m_per, n_per), jnp.float32),
        in_specs=[
            pl.BlockSpec(memory_space=pltpu.VMEM),
            pl.BlockSpec(memory_space=pltpu.VMEM),
            pl.BlockSpec(memory_space=pltpu.VMEM),
            pl.BlockSpec(memory_space=pltpu.VMEM),
        ],
        out_specs=pl.BlockSpec(memory_space=pltpu.VMEM),
        scratch_shapes=[
            pltpu.VMEM((N_DEV, k, n_per), jnp.int8),
            pltpu.VMEM((N_DEV - 1, m_per, n_per), jnp.int16),
            pltpu.VMEM((N_DEV - 1, m_per, n_per), jnp.int16),
            pltpu.SemaphoreType.DMA((N_DEV - 1,)),
            pltpu.SemaphoreType.DMA((N_DEV - 1,)),
        ],
        compiler_params=pltpu.CompilerParams(collective_id=0),
    )(x, w_mat, scale_x, scale_w)


# device time: 45882 ns/iter; 1.0136x vs baseline; 1.0136x over previous
import jax
import jax.numpy as jnp
from jax import lax
from jax.experimental import pallas as pl
from jax.experimental.pallas import tpu as pltpu

N_DEV = 4


def kernel(x, w_mat, scale_x, scale_w):
    m_per, k = x.shape
    _, n = w_mat.shape
    n_per = n // N_DEV

    def body(x_ref, w_ref, sx_ref, sw_ref, out_ref,
             ybuf, rbuf, send_sems, recv_sems):
        my = lax.axis_index("i")

        barrier_sem = pltpu.get_barrier_semaphore()
        for d in range(1, N_DEV):
            pl.semaphore_signal(
                barrier_sem,
                inc=1,
                device_id=((my + d) % N_DEV,),
                device_id_type=pl.DeviceIdType.MESH,
            )
        pl.semaphore_wait(barrier_sem, N_DEV - 1)

        scale = sx_ref[0] * sw_ref[0]

        for d in range(1, N_DEV):
            p = (my + d) % N_DEV
            acc = jnp.dot(x_ref[:, pl.ds(0, k)],
                          w_ref[:, pl.ds(p * n_per, n_per)],
                          preferred_element_type=jnp.int32)
            ybuf[d - 1, :, :] = ((acc + 1024) >> 11).astype(jnp.int16)
            rdma = pltpu.make_async_remote_copy(
                src_ref=ybuf.at[d - 1],
                dst_ref=rbuf.at[d - 1],
                send_sem=send_sems.at[d - 1],
                recv_sem=recv_sems.at[d - 1],
                device_id=(p,),
                device_id_type=pl.DeviceIdType.MESH,
            )
            rdma.start()

        acc = jnp.dot(x_ref[:, :], w_ref[:, pl.ds(my * n_per, n_per)],
                      preferred_element_type=jnp.int32)
        out_ref[pl.ds(my * m_per, m_per), :] = acc.astype(jnp.float32) * scale

        for d in range(1, N_DEV):
            src = (my - d) % N_DEV
            waiter = pltpu.make_async_remote_copy(
                src_ref=ybuf.at[d - 1],
                dst_ref=rbuf.at[d - 1],
                send_sem=send_sems.at[d - 1],
                recv_sem=recv_sems.at[d - 1],
                device_id=((my + d) % N_DEV,),
                device_id_type=pl.DeviceIdType.MESH,
            )
            waiter.wait_recv()
            out_ref[pl.ds(src * m_per, m_per), :] = (
                rbuf[d - 1, :, :].astype(jnp.float32) * (2048.0 * scale))
            waiter.wait_send()

    return pl.pallas_call(
        body,
        out_shape=jax.ShapeDtypeStruct((N_DEV * m_per, n_per), jnp.float32),
        in_specs=[
            pl.BlockSpec(memory_space=pltpu.VMEM),
            pl.BlockSpec(memory_space=pltpu.VMEM),
            pl.BlockSpec(memory_space=pltpu.VMEM),
            pl.BlockSpec(memory_space=pltpu.VMEM),
        ],
        out_specs=pl.BlockSpec(memory_space=pltpu.VMEM),
        scratch_shapes=[
            pltpu.VMEM((N_DEV - 1, m_per, n_per), jnp.int16),
            pltpu.VMEM((N_DEV - 1, m_per, n_per), jnp.int16),
            pltpu.SemaphoreType.DMA((N_DEV - 1,)),
            pltpu.SemaphoreType.DMA((N_DEV - 1,)),
        ],
        compiler_params=pltpu.CompilerParams(collective_id=0),
    )(x, w_mat, scale_x, scale_w)
